# baseline (speedup 1.0000x reference)
import functools

import jax
import jax.numpy as jnp
from jax import lax
from jax.experimental import pallas as pl
from jax.experimental.pallas import tpu as pltpu
from jax.experimental.pallas import tpu_sc as plsc

L = 16


def _make_scatter_sorted(P, N, M):
    info = plsc.get_sparse_core_info()
    nc, ns = info.num_cores, info.num_subcores
    nw = nc * ns
    assert P % nw == 0
    pp = P // nw

    mesh = plsc.VectorSubcoreMesh(core_axis_name="c", subcore_axis_name="s")

    @functools.partial(
        pl.kernel,
        mesh=mesh,
        compiler_params=pltpu.CompilerParams(needs_layout_passes=False),
        out_type=jax.ShapeDtypeStruct((P, M), jnp.float32),
        scratch_types=[
            pltpu.VMEM((N + L,), jnp.int32),
            pltpu.VMEM((N,), jnp.float32),
            pltpu.VMEM((M,), jnp.float32),
        ],
    )
    def k(skey_hbm, sval_hbm, out_hbm, ibuf, xbuf, obuf):
        wid = lax.axis_index("s") * nc + lax.axis_index("c")
        zeros = jnp.zeros((L,), jnp.float32)

        def zbody(i, c):
            obuf[pl.ds(i * L, L)] = zeros
            return c

        lax.fori_loop(0, M // L, zbody, 0)
        ibuf[pl.ds(N, L)] = jnp.full((L,), -1, jnp.int32)

        def plane_body(j, c):
            p = wid * pp + j
            base = p * M
            pltpu.sync_copy(skey_hbm.at[pl.ds(p * N, N)], ibuf.at[pl.ds(0, N)])
            pltpu.sync_copy(sval_hbm.at[pl.ds(p * N, N)], xbuf)

            def sbody(i, c):
                cur = ibuf[pl.ds(i * L, L)]
                nxt = ibuf[pl.ds(i * L + 1, L)]
                loc = cur - base
                xv = xbuf[pl.ds(i * L, L)]
                plsc.store_scatter(obuf, [loc], xv, mask=cur != nxt)
                return c

            lax.fori_loop(0, N // L, sbody, 0)
            pltpu.sync_copy(obuf, out_hbm.at[p])

            def zsbody(i, c):
                loc = ibuf[pl.ds(i * L, L)] - base
                plsc.store_scatter(obuf, [loc], zeros)
                return c

            lax.fori_loop(0, N // L, zsbody, 0)
            return c

        lax.fori_loop(0, pp, plane_body, 0)

    return k


def kernel(x, idx, x1):
    B, C, Hp, Wp = x.shape
    _, _, H, W = x1.shape
    P, N, M = B * C, Hp * Wp, H * W
    planes = jnp.arange(P, dtype=jnp.int32)[:, None] * jnp.int32(M)
    keys = (idx.reshape(P, N) + planes).reshape(P * N)
    vals = x.reshape(P * N)
    skeys, svals = lax.sort((keys, vals), dimension=0, num_keys=1, is_stable=False)
    scatter = _make_scatter_sorted(P, N, M)
    out2 = scatter(skeys, svals)
    return out2.reshape(B, C, H, W)


# speedup vs baseline: 1.0271x; 1.0271x over previous
import functools

import jax
import jax.numpy as jnp
from jax import lax
from jax.experimental import pallas as pl
from jax.experimental.pallas import tpu as pltpu
from jax.experimental.pallas import tpu_sc as plsc

L = 16


def _make_scatter_sorted(P, N, M):
    info = plsc.get_sparse_core_info()
    nc, ns = info.num_cores, info.num_subcores
    nw = nc * ns
    assert P % nw == 0
    pp = P // nw

    mesh = plsc.VectorSubcoreMesh(core_axis_name="c", subcore_axis_name="s")

    @functools.partial(
        pl.kernel,
        mesh=mesh,
        compiler_params=pltpu.CompilerParams(needs_layout_passes=False),
        out_type=jax.ShapeDtypeStruct((P, M), jnp.float32),
        scratch_types=[
            pltpu.VMEM((2, N + L), jnp.int32),
            pltpu.VMEM((2, N), jnp.float32),
            pltpu.VMEM((M,), jnp.float32),
            pltpu.SemaphoreType.DMA,
            pltpu.SemaphoreType.DMA,
        ],
    )
    def k(skey_hbm, sval_hbm, out_hbm, ibuf, xbuf, obuf, semk, semv):
        wid = lax.axis_index("s") * nc + lax.axis_index("c")
        p0 = wid * pp
        zeros = jnp.zeros((L,), jnp.float32)

        @plsc.parallel_loop(0, M // L, unroll=8)
        def _(i):
            obuf[pl.ds(i * L, L)] = zeros

        sentinel = jnp.full((L,), -1, jnp.int32)
        ibuf[0, pl.ds(N, L)] = sentinel
        ibuf[1, pl.ds(N, L)] = sentinel

        pltpu.async_copy(skey_hbm.at[pl.ds(p0 * N, N)], ibuf.at[0, pl.ds(0, N)], semk)
        pltpu.async_copy(sval_hbm.at[pl.ds(p0 * N, N)], xbuf.at[0], semv)

        def plane_body(j, c):
            s = lax.rem(j, 2)
            p = p0 + j
            base = p * M
            pltpu.make_async_copy(
                skey_hbm.at[pl.ds(p * N, N)], ibuf.at[s, pl.ds(0, N)], semk
            ).wait()
            pltpu.make_async_copy(
                sval_hbm.at[pl.ds(p * N, N)], xbuf.at[s], semv
            ).wait()

            @pl.when(j + 1 < pp)
            def _():
                o = 1 - s
                pltpu.async_copy(
                    skey_hbm.at[pl.ds((p + 1) * N, N)], ibuf.at[o, pl.ds(0, N)], semk
                )
                pltpu.async_copy(
                    sval_hbm.at[pl.ds((p + 1) * N, N)], xbuf.at[o], semv
                )

            @plsc.parallel_loop(0, N // L, unroll=8)
            def _(i):
                cur = ibuf[s, pl.ds(i * L, L)]
                nxt = ibuf[s, pl.ds(i * L + 1, L)]
                xv = xbuf[s, pl.ds(i * L, L)]
                plsc.store_scatter(obuf, [cur - base], xv, mask=cur != nxt)

            pltpu.sync_copy(obuf, out_hbm.at[p])

            @plsc.parallel_loop(0, N // L, unroll=8)
            def _(i):
                loc = ibuf[s, pl.ds(i * L, L)] - base
                plsc.store_scatter(obuf, [loc], zeros)

            return c

        lax.fori_loop(0, pp, plane_body, 0)

    return k


def kernel(x, idx, x1):
    B, C, Hp, Wp = x.shape
    _, _, H, W = x1.shape
    P, N, M = B * C, Hp * Wp, H * W
    planes = jnp.arange(P, dtype=jnp.int32)[:, None] * jnp.int32(M)
    keys = (idx.reshape(P, N) + planes).reshape(P * N)
    vals = x.reshape(P * N)
    skeys, svals = lax.sort((keys, vals), dimension=0, num_keys=1, is_stable=False)
    scatter = _make_scatter_sorted(P, N, M)
    out2 = scatter(skeys, svals)
    return out2.reshape(B, C, H, W)
